# Initial kernel scaffold; baseline (speedup 1.0000x reference)
import jax
import jax.numpy as jnp
from jax.experimental import pallas as pl


def kernel(x, edge_index0, edge_index1, size0_dst, size1_dst, W_l0, W_r0, b0, W_l1, W_r1, b1):
    raise NotImplementedError("write your pallas kernel here")


# speedup vs baseline: 3.3091x; 3.3091x over previous
import functools

import jax
import jax.numpy as jnp
from jax import lax
from jax.experimental import pallas as pl
from jax.experimental.pallas import tpu as pltpu
from jax.experimental.pallas import tpu_sc as plsc

_IN_CH = 128
_HID = 128
_OUT = 64
_N_SRC = 100000
_ND0 = 20000
_ND1 = 4096
_E0 = 320000
_E1 = 65536
_NC = 2
_NS = 16

_HALF0 = _ND0 // 2
_ROWS0 = 10240
_GARB0 = _HALF0
_CH0 = 80
_EPT0 = _E0 // _NS
_NCH0 = _EPT0 // _CH0

_CH1 = 128
_EPC1 = _E1 // _NC
_EPT1 = _EPC1 // _NS
_NCH1 = _EPT1 // _CH1

_CBLK = 2000
_BBLK = 1000
_HBLK = 400
_FBLK = 512


def _colsum_body(x_ref, out_ref):
    i = pl.program_id(0)

    @pl.when(i == 0)
    def _():
        out_ref[...] = jnp.zeros_like(out_ref)

    ssum = jnp.sum(x_ref[...], axis=0, keepdims=True)
    out_ref[0:1, :] += ssum

    @pl.when(i < _ND0 // _CBLK)
    def _():
        out_ref[1:2, :] += ssum


def _binarize_body(sums_ref, x_ref, xb_ref, xtb_ref, *, nfull, npart):
    m_full = sums_ref[0:1, :] / nfull
    m_part = sums_ref[1:2, :] / npart
    blk = x_ref[...]
    xb_ref[...] = jnp.sign(blk - m_full)
    xtb_ref[...] = jnp.sign(blk - m_part)


def _layer0_body(agg_ref, cnt_ref, xtb_ref, wl_ref, wr_ref, b_ref,
                 h_ref, hsums_ref):
    i = pl.program_id(0)

    @pl.when(i == 0)
    def _():
        hsums_ref[...] = jnp.zeros_like(hsums_ref)

    cnt = jnp.maximum(cnt_ref[...], 1.0)
    ma = agg_ref[...] / cnt
    hblk = (jnp.dot(ma, wl_ref[...], preferred_element_type=jnp.float32)
            + jnp.dot(xtb_ref[...], wr_ref[...], preferred_element_type=jnp.float32)
            + b_ref[...])
    hblk = jnp.maximum(hblk, 0.0)
    h_ref[...] = hblk
    hsums_ref[0:1, :] += jnp.sum(hblk, axis=0, keepdims=True)
    rows = i * _HBLK + lax.broadcasted_iota(jnp.int32, (_HBLK, 1), 0)
    hsums_ref[1:2, :] += jnp.sum(jnp.where(rows < _ND1, hblk, 0.0),
                                 axis=0, keepdims=True)


def _final_body(aggA_ref, aggB_ref, cntA_ref, cntB_ref, htb_ref,
                wl_ref, wr_ref, b_ref, out_ref):
    cnt = jnp.maximum(cntA_ref[...] + cntB_ref[...], 1.0)
    ma = (aggA_ref[...] + aggB_ref[...]) / cnt
    z = (jnp.dot(ma, wl_ref[...], preferred_element_type=jnp.float32)
         + jnp.dot(htb_ref[...], wr_ref[...], preferred_element_type=jnp.float32)
         + b_ref[...])
    z = z - jnp.max(z, axis=1, keepdims=True)
    z = z - jnp.log(jnp.sum(jnp.exp(z), axis=1, keepdims=True))
    out_ref[...] = z


def _sc_agg0_body(xb, src, dst, zr, agg_out, cnt_out,
                  src_v, dst_v, sidx_v, rows_v, ones_v, cstage_v,
                  agg_sh, cnt_sh, sem):
    c = lax.axis_index("c")
    s = lax.axis_index("s")
    base = c * _HALF0

    def zbody(i, carry):
        cstage_v[pl.ds(i * 16, 16)] = jnp.zeros((16,), jnp.float32)
        return carry

    lax.fori_loop(0, 1008 // 16, zbody, 0)
    pltpu.sync_copy(zr.at[pl.ds(s * 640, 640)], agg_sh.at[pl.ds(s * 640, 640)])
    pltpu.sync_copy(cstage_v.at[pl.ds(0, 640)], cnt_sh.at[pl.ds(s * 640, 640)])
    for k in range(_CH0 // 16):
        ones_v[pl.ds(k * 16, 16)] = jnp.full((16,), 1.0, jnp.float32)
    plsc.subcore_barrier()

    def body(i, carry):
        ebase = s * _EPT0 + i * _CH0
        pltpu.sync_copy(src.at[pl.ds(ebase, _CH0)], src_v)
        pltpu.sync_copy(dst.at[pl.ds(ebase, _CH0)], dst_v)
        for k in range(_CH0 // 16):
            d = dst_v[pl.ds(k * 16, 16)]
            loc = d - base
            oob = (loc < 0) | (loc >= _HALF0)
            sidx_v[pl.ds(k * 16, 16)] = jnp.where(oob, _GARB0, loc)
        pltpu.async_copy(xb.at[src_v], rows_v, sem).wait()
        pltpu.sync_copy(rows_v, agg_sh.at[sidx_v], add=True)
        pltpu.sync_copy(ones_v, cnt_sh.at[sidx_v], add=True)
        return carry

    lax.fori_loop(0, _NCH0, body, 0)
    plsc.subcore_barrier()

    @pl.when(s < 10)
    def _():
        pltpu.sync_copy(agg_sh.at[pl.ds(s * 1000, 1000)],
                        agg_out.at[pl.ds(c * _HALF0 + s * 1000, 1000)])
        pltpu.sync_copy(cnt_sh.at[pl.ds(s * 1000, 1000)],
                        cstage_v.at[pl.ds(0, 1000)])
        pltpu.sync_copy(cstage_v.at[pl.ds(0, 1000)],
                        cnt_out.at[pl.ds(c * _HALF0 + s * 1000, 1000)])


def _sc_agg1_body(hb, src, dst, zr, agg_out, cnt_out,
                  src_v, dst_v, rows_v, ones_v, cstage_v, agg_sh, cnt_sh, sem):
    c = lax.axis_index("c")
    s = lax.axis_index("s")

    def zbody(i, carry):
        cstage_v[pl.ds(i * 16, 16)] = jnp.zeros((16,), jnp.float32)
        return carry

    lax.fori_loop(0, 256 // 16, zbody, 0)
    pltpu.sync_copy(zr.at[pl.ds(s * 256, 256)], agg_sh.at[pl.ds(s * 256, 256)])
    pltpu.sync_copy(cstage_v, cnt_sh.at[pl.ds(s * 256, 256)])
    for k in range(_CH1 // 16):
        ones_v[pl.ds(k * 16, 16)] = jnp.full((16,), 1.0, jnp.float32)
    plsc.subcore_barrier()

    def body(i, carry):
        ebase = c * _EPC1 + s * _EPT1 + i * _CH1
        pltpu.sync_copy(src.at[pl.ds(ebase, _CH1)], src_v)
        pltpu.sync_copy(dst.at[pl.ds(ebase, _CH1)], dst_v)
        pltpu.async_copy(hb.at[src_v], rows_v, sem).wait()
        pltpu.sync_copy(rows_v, agg_sh.at[dst_v], add=True)
        pltpu.sync_copy(ones_v, cnt_sh.at[dst_v], add=True)
        return carry

    lax.fori_loop(0, _NCH1, body, 0)
    plsc.subcore_barrier()
    pltpu.sync_copy(agg_sh.at[pl.ds(s * 256, 256)],
                    agg_out.at[pl.ds(c * _ND1 + s * 256, 256)])
    pltpu.sync_copy(cnt_sh.at[pl.ds(s * 256, 256)], cstage_v)
    pltpu.sync_copy(cstage_v, cnt_out.at[pl.ds(c * _ND1 + s * 256, 256)])


@functools.cache
def _sc_kernels():
    mesh = plsc.VectorSubcoreMesh(core_axis_name="c", subcore_axis_name="s",
                                  num_cores=_NC, num_subcores=_NS)
    agg0 = pl.kernel(
        _sc_agg0_body,
        out_type=[jax.ShapeDtypeStruct((_ND0, _IN_CH), jnp.float32),
                  jax.ShapeDtypeStruct((_ND0,), jnp.float32)],
        mesh=mesh,
        scratch_types=[
            pltpu.VMEM((_CH0,), jnp.int32),
            pltpu.VMEM((_CH0,), jnp.int32),
            pltpu.VMEM((_CH0,), jnp.int32),
            pltpu.VMEM((_CH0, _IN_CH), jnp.float32),
            pltpu.VMEM((_CH0,), jnp.float32),
            pltpu.VMEM((1008,), jnp.float32),
            pltpu.VMEM_SHARED((_ROWS0, _IN_CH), jnp.float32),
            pltpu.VMEM_SHARED((_ROWS0,), jnp.float32),
            pltpu.SemaphoreType.DMA,
        ],
    )
    agg1 = pl.kernel(
        _sc_agg1_body,
        out_type=[jax.ShapeDtypeStruct((_NC * _ND1, _HID), jnp.float32),
                  jax.ShapeDtypeStruct((_NC * _ND1,), jnp.float32)],
        mesh=mesh,
        scratch_types=[
            pltpu.VMEM((_CH1,), jnp.int32),
            pltpu.VMEM((_CH1,), jnp.int32),
            pltpu.VMEM((_CH1, _HID), jnp.float32),
            pltpu.VMEM((_CH1,), jnp.float32),
            pltpu.VMEM((256,), jnp.float32),
            pltpu.VMEM_SHARED((_ND1, _HID), jnp.float32),
            pltpu.VMEM_SHARED((_ND1,), jnp.float32),
            pltpu.SemaphoreType.DMA,
        ],
    )
    return agg0, agg1


def kernel(x, edge_index0, edge_index1, size0_dst, size1_dst,
           W_l0, W_r0, b0, W_l1, W_r1, b1):
    f32 = jnp.float32
    x = x.astype(f32)
    src0 = edge_index0[0]
    dst0 = edge_index0[1]
    src1 = edge_index1[0]
    dst1 = edge_index1[1]

    sums = pl.pallas_call(
        _colsum_body,
        grid=(_N_SRC // _CBLK,),
        in_specs=[pl.BlockSpec((_CBLK, _IN_CH), lambda i: (i, 0))],
        out_specs=pl.BlockSpec((8, _IN_CH), lambda i: (0, 0)),
        out_shape=jax.ShapeDtypeStruct((8, _IN_CH), f32),
    )(x)

    xb, xtb = pl.pallas_call(
        functools.partial(_binarize_body, nfull=float(_N_SRC), npart=float(_ND0)),
        grid=(_ND0 // _BBLK,),
        in_specs=[pl.BlockSpec((8, _IN_CH), lambda i: (0, 0)),
                  pl.BlockSpec((_BBLK, _IN_CH), lambda i: (i, 0))],
        out_specs=[pl.BlockSpec((_BBLK, _IN_CH), lambda i: (i, 0))] * 2,
        out_shape=[jax.ShapeDtypeStruct((_ND0, _IN_CH), f32)] * 2,
    )(sums, x)

    zr = jnp.zeros((_ROWS0, _IN_CH), f32)
    sc_agg0, sc_agg1 = _sc_kernels()
    agg0, cnt0 = sc_agg0(xb, src0, dst0, zr)

    h, hsums = pl.pallas_call(
        _layer0_body,
        grid=(_ND0 // _HBLK,),
        in_specs=[pl.BlockSpec((_HBLK, _IN_CH), lambda i: (i, 0)),
                  pl.BlockSpec((_HBLK, 1), lambda i: (i, 0)),
                  pl.BlockSpec((_HBLK, _IN_CH), lambda i: (i, 0)),
                  pl.BlockSpec((_IN_CH, _HID), lambda i: (0, 0)),
                  pl.BlockSpec((_IN_CH, _HID), lambda i: (0, 0)),
                  pl.BlockSpec((1, _HID), lambda i: (0, 0))],
        out_specs=[pl.BlockSpec((_HBLK, _HID), lambda i: (i, 0)),
                   pl.BlockSpec((8, _HID), lambda i: (0, 0))],
        out_shape=[jax.ShapeDtypeStruct((_ND0, _HID), f32),
                   jax.ShapeDtypeStruct((8, _HID), f32)],
    )(agg0, cnt0.reshape(_ND0, 1), xtb, W_l0, W_r0, b0.reshape(1, _HID))

    hb, htb = pl.pallas_call(
        functools.partial(_binarize_body, nfull=float(_ND0), npart=float(_ND1)),
        grid=(_ND1 // _FBLK,),
        in_specs=[pl.BlockSpec((8, _HID), lambda i: (0, 0)),
                  pl.BlockSpec((_FBLK, _HID), lambda i: (i, 0))],
        out_specs=[pl.BlockSpec((_FBLK, _HID), lambda i: (i, 0))] * 2,
        out_shape=[jax.ShapeDtypeStruct((_ND1, _HID), f32)] * 2,
    )(hsums, h)

    agg1p, cnt1p = sc_agg1(hb, src1, dst1, zr)

    out = pl.pallas_call(
        _final_body,
        grid=(_ND1 // _FBLK,),
        in_specs=[pl.BlockSpec((_FBLK, _HID), lambda i: (i, 0)),
                  pl.BlockSpec((_FBLK, _HID), lambda i: (i, 0)),
                  pl.BlockSpec((_FBLK, 1), lambda i: (i, 0)),
                  pl.BlockSpec((_FBLK, 1), lambda i: (i, 0)),
                  pl.BlockSpec((_FBLK, _HID), lambda i: (i, 0)),
                  pl.BlockSpec((_HID, _OUT), lambda i: (0, 0)),
                  pl.BlockSpec((_HID, _OUT), lambda i: (0, 0)),
                  pl.BlockSpec((1, _OUT), lambda i: (0, 0))],
        out_specs=pl.BlockSpec((_FBLK, _OUT), lambda i: (i, 0)),
        out_shape=jax.ShapeDtypeStruct((_ND1, _OUT), f32),
    )(agg1p[:_ND1], agg1p[_ND1:], cnt1p[:_ND1].reshape(_ND1, 1),
      cnt1p[_ND1:].reshape(_ND1, 1), htb, W_l1, W_r1, b1.reshape(1, _OUT))
    return out
